# baseline (speedup 1.0000x reference)
import jax
import jax.numpy as jnp
from jax.experimental import pallas as pl


def _product_body(x_ref, o_ref):
    x = x_ref[...]
    bB, F2, _, CR = x.shape
    C = o_ref.shape[2]
    R = CR // C
    left = x[:, :, 0, :].reshape(bB, F2, C, 1, R)
    right = x[:, :, 1, :].reshape(bB, F2, 1, C, R)
    out = left + right
    o_ref[...] = out.reshape(bB, F2, C, CR)


def kernel(log_likelihoods):
    B, F, C, R = log_likelihoods.shape
    F2 = F // 2
    x = log_likelihoods.reshape(B, F2, 2, C * R)
    bB = 1
    out = pl.pallas_call(
        _product_body,
        grid=(B // bB,),
        in_specs=[pl.BlockSpec((bB, F2, 2, C * R), lambda b: (b, 0, 0, 0))],
        out_specs=pl.BlockSpec((bB, F2, C, C * R), lambda b: (b, 0, 0, 0)),
        out_shape=jax.ShapeDtypeStruct((B, F2, C, C * R), jnp.float32),
    )(x)
    return out.reshape(B, F2, C * C, R)


# speedup vs baseline: 1.4251x; 1.4251x over previous
import jax
import jax.numpy as jnp
from jax.experimental import pallas as pl
from jax.experimental.pallas import tpu as pltpu


def _product_body(xl_ref, xr_ref, o_ref):
    l = xl_ref[0, :, 0, :, :]
    r = xr_ref[0, :, 1, :]
    C = l.shape[1]
    lt = jnp.concatenate([l] * C, axis=-1)
    o_ref[0] = lt + r[:, None, :]


def kernel(log_likelihoods):
    B, F, C, R = log_likelihoods.shape
    F2 = F // 2
    xl = log_likelihoods.reshape(B, F2, 2, C, R)
    xr = log_likelihoods.reshape(B, F2, 2, C * R)
    out = pl.pallas_call(
        _product_body,
        grid=(B,),
        in_specs=[
            pl.BlockSpec((1, F2, 1, C, R), lambda b: (b, 0, 0, 0, 0)),
            pl.BlockSpec((1, F2, 2, C * R), lambda b: (b, 0, 0, 0)),
        ],
        out_specs=pl.BlockSpec((1, F2, C, C * R), lambda b: (b, 0, 0, 0)),
        out_shape=jax.ShapeDtypeStruct((B, F2, C, C * R), jnp.float32),
        compiler_params=pltpu.CompilerParams(
            dimension_semantics=("arbitrary",),
        ),
    )(xl, xr)
    return out.reshape(B, F2, C * C, R)


# speedup vs baseline: 2.2420x; 1.5732x over previous
import numpy as np
import jax
import jax.numpy as jnp
from jax.experimental import pallas as pl
from jax.experimental.pallas import tpu as pltpu


def _build_selector(C: int, R: int) -> np.ndarray:
    CR = C * R
    o = np.arange(C * CR)
    i = o // CR
    m = o % CR
    r = m % R
    w = np.zeros((2 * CR, C * CR), np.float32)
    w[i * R + r, o] = 1.0
    w[CR + m, o] = 1.0
    return w


def _product_body(x_ref, w_ref, o_ref):
    o_ref[...] = jnp.dot(
        x_ref[...], w_ref[...], preferred_element_type=jnp.float32
    )


def kernel(log_likelihoods):
    B, F, C, R = log_likelihoods.shape
    F2 = F // 2
    CR = C * R
    N = B * F2
    x2 = log_likelihoods.reshape(N, 2 * CR)
    w = jnp.asarray(_build_selector(C, R))
    bM = 1024
    out = pl.pallas_call(
        _product_body,
        grid=(N // bM,),
        in_specs=[
            pl.BlockSpec((bM, 2 * CR), lambda g: (g, 0)),
            pl.BlockSpec((2 * CR, C * CR), lambda g: (0, 0)),
        ],
        out_specs=pl.BlockSpec((bM, C * CR), lambda g: (g, 0)),
        out_shape=jax.ShapeDtypeStruct((N, C * CR), jnp.float32),
        compiler_params=pltpu.CompilerParams(
            dimension_semantics=("arbitrary",),
        ),
    )(x2, w)
    return out.reshape(B, F2, C * C, R)


# speedup vs baseline: 12.9020x; 5.7546x over previous
import numpy as np
import jax
import jax.numpy as jnp
from jax import lax
from jax.experimental import pallas as pl
from jax.experimental.pallas import tpu as pltpu


def _build_selectors(C: int, R: int) -> np.ndarray:
    CR, CC = C * R, C * C
    m = np.arange(CC)
    i = m // C
    j = m % C
    w = np.zeros((2 * R, CR, CC), np.float32)
    for r in range(R):
        w[2 * r + 0, i * R + r, m] = 1.0
        w[2 * r + 1, j * R + r, m] = 1.0
    return w


def _product_body(xt_ref, w_ref, e_ref, o_ref):
    C, R, F = xt_ref.shape[1:]
    x64 = xt_ref[0].reshape(C * R, F)
    xe = jnp.dot(x64, e_ref[0], preferred_element_type=jnp.float32)
    xo = jnp.dot(x64, e_ref[1], preferred_element_type=jnp.float32)
    dn = (((0,), (0,)), ((), ()))
    for r in range(R):
        p = lax.dot_general(xe, w_ref[2 * r + 0],
                            dn, preferred_element_type=jnp.float32)
        q = lax.dot_general(xo, w_ref[2 * r + 1],
                            dn, preferred_element_type=jnp.float32)
        o_ref[0, :, r, :] = p + q


def kernel(log_likelihoods):
    B, F, C, R = log_likelihoods.shape
    F2 = F // 2
    CR, CC = C * R, C * C
    xt = jnp.transpose(log_likelihoods, (0, 2, 3, 1))
    w = jnp.asarray(_build_selectors(C, R))
    e = np.zeros((2, F, F2), np.float32)
    e[0, 2 * np.arange(F2), np.arange(F2)] = 1.0
    e[1, 2 * np.arange(F2) + 1, np.arange(F2)] = 1.0
    e = jnp.asarray(e)
    yt = pl.pallas_call(
        _product_body,
        grid=(B,),
        in_specs=[
            pl.BlockSpec((1, C, R, F), lambda b: (b, 0, 0, 0)),
            pl.BlockSpec((2 * R, CR, CC), lambda b: (0, 0, 0)),
            pl.BlockSpec((2, F, F2), lambda b: (0, 0, 0)),
        ],
        out_specs=pl.BlockSpec((1, F2, R, CC), lambda b: (b, 0, 0, 0)),
        out_shape=jax.ShapeDtypeStruct((B, F2, R, CC), jnp.float32),
        compiler_params=pltpu.CompilerParams(
            dimension_semantics=("arbitrary",),
        ),
    )(xt, w, e)
    return jnp.transpose(yt, (0, 1, 3, 2))


# speedup vs baseline: 21.4194x; 1.6602x over previous
import numpy as np
import jax
import jax.numpy as jnp
from jax import lax
from jax.experimental import pallas as pl
from jax.experimental.pallas import tpu as pltpu


def _build_selectors(C: int, R: int) -> np.ndarray:
    CR, CC = C * R, C * C
    m = np.arange(CC)
    i = m // C
    j = m % C
    w = np.zeros((2 * R, CR, CC), np.float32)
    for r in range(R):
        w[2 * r + 0, i * R + r, m] = 1.0
        w[2 * r + 1, j * R + r, m] = 1.0
    return w


def _product_body(xt_ref, w_ref, e_ref, o_ref):
    bB = xt_ref.shape[0]
    C, R, F = xt_ref.shape[1:]
    dn = (((0,), (0,)), ((), ()))
    for b in range(bB):
        x64 = xt_ref[b].reshape(C * R, F)
        xe = jnp.dot(x64, e_ref[0], preferred_element_type=jnp.float32)
        xo = jnp.dot(x64, e_ref[1], preferred_element_type=jnp.float32)
        for r in range(R):
            p = lax.dot_general(xe, w_ref[2 * r + 0],
                                dn, preferred_element_type=jnp.float32)
            q = lax.dot_general(xo, w_ref[2 * r + 1],
                                dn, preferred_element_type=jnp.float32)
            o_ref[b, :, r, :] = p + q


def kernel(log_likelihoods):
    B, F, C, R = log_likelihoods.shape
    F2 = F // 2
    CR, CC = C * R, C * C
    xt = jnp.transpose(log_likelihoods, (0, 2, 3, 1))
    w = jnp.asarray(_build_selectors(C, R))
    e = np.zeros((2, F, F2), np.float32)
    e[0, 2 * np.arange(F2), np.arange(F2)] = 1.0
    e[1, 2 * np.arange(F2) + 1, np.arange(F2)] = 1.0
    e = jnp.asarray(e)
    bB = 4
    yt = pl.pallas_call(
        _product_body,
        grid=(B // bB,),
        in_specs=[
            pl.BlockSpec((bB, C, R, F), lambda b: (b, 0, 0, 0)),
            pl.BlockSpec((2 * R, CR, CC), lambda b: (0, 0, 0)),
            pl.BlockSpec((2, F, F2), lambda b: (0, 0, 0)),
        ],
        out_specs=pl.BlockSpec((bB, F2, R, CC), lambda b: (b, 0, 0, 0)),
        out_shape=jax.ShapeDtypeStruct((B, F2, R, CC), jnp.float32),
        compiler_params=pltpu.CompilerParams(
            dimension_semantics=("arbitrary",),
        ),
    )(xt, w, e)
    return jnp.transpose(yt, (0, 1, 3, 2))


# speedup vs baseline: 21.6588x; 1.0112x over previous
import numpy as np
import jax
import jax.numpy as jnp
from jax import lax
from jax.experimental import pallas as pl
from jax.experimental.pallas import tpu as pltpu


def _build_selectors(C: int, R: int) -> np.ndarray:
    CR, CC = C * R, C * C
    m = np.arange(CC)
    i = m // C
    j = m % C
    w = np.zeros((2 * R, CR, CC), np.float32)
    for r in range(R):
        w[2 * r + 0, i * R + r, m] = 1.0
        w[2 * r + 1, j * R + r, m] = 1.0
    return w


def _product_body(xt_ref, w_ref, e_ref, o_ref):
    bB = xt_ref.shape[0]
    C, R, F = xt_ref.shape[1:]
    dn = (((0,), (0,)), ((), ()))
    for b in range(bB):
        x64 = xt_ref[b].reshape(C * R, F)
        xe = jnp.dot(x64, e_ref[0], preferred_element_type=jnp.float32)
        xo = jnp.dot(x64, e_ref[1], preferred_element_type=jnp.float32)
        for r in range(R):
            p = lax.dot_general(xe, w_ref[2 * r + 0],
                                dn, preferred_element_type=jnp.float32)
            q = lax.dot_general(xo, w_ref[2 * r + 1],
                                dn, preferred_element_type=jnp.float32)
            o_ref[b, :, r, :] = p + q


def kernel(log_likelihoods):
    B, F, C, R = log_likelihoods.shape
    F2 = F // 2
    CR, CC = C * R, C * C
    xt = jnp.transpose(log_likelihoods, (0, 2, 3, 1))
    w = jnp.asarray(_build_selectors(C, R))
    e = np.zeros((2, F, F2), np.float32)
    e[0, 2 * np.arange(F2), np.arange(F2)] = 1.0
    e[1, 2 * np.arange(F2) + 1, np.arange(F2)] = 1.0
    e = jnp.asarray(e)
    bB = 8
    yt = pl.pallas_call(
        _product_body,
        grid=(B // bB,),
        in_specs=[
            pl.BlockSpec((bB, C, R, F), lambda b: (b, 0, 0, 0)),
            pl.BlockSpec((2 * R, CR, CC), lambda b: (0, 0, 0)),
            pl.BlockSpec((2, F, F2), lambda b: (0, 0, 0)),
        ],
        out_specs=pl.BlockSpec((bB, F2, R, CC), lambda b: (b, 0, 0, 0)),
        out_shape=jax.ShapeDtypeStruct((B, F2, R, CC), jnp.float32),
        compiler_params=pltpu.CompilerParams(
            dimension_semantics=("arbitrary",),
        ),
    )(xt, w, e)
    return jnp.transpose(yt, (0, 1, 3, 2))


# speedup vs baseline: 28.0636x; 1.2957x over previous
import numpy as np
import jax
import jax.numpy as jnp
from jax import lax
from jax.experimental import pallas as pl
from jax.experimental.pallas import tpu as pltpu


def _build_consts(C: int, R: int, F: int):
    F2 = F // 2
    CR, CC = C * R, C * C
    m = np.arange(CC)
    i, j = m // C, m % C
    k = np.arange(CR)
    w = np.zeros((2, CR, CC), np.float32)
    w[0, :, :] = (k[:, None] // R == i[None, :])
    w[1, :, :] = (k[:, None] // R == j[None, :])
    mm = np.arange(R * F2)
    e = np.zeros((2, F, R * F2), np.float32)
    e[0, 2 * (mm // R), mm] = 1.0
    e[1, 2 * (mm // R) + 1, mm] = 1.0
    msk = (k[:, None] % R == mm[None, :] % R).astype(np.float32)
    return jnp.asarray(w), jnp.asarray(e), jnp.asarray(msk)


def _product_body(xt_ref, w_ref, e_ref, m_ref, o_ref):
    bB = xt_ref.shape[0]
    C, R, F = xt_ref.shape[1:]
    F2 = F // 2
    dn = (((0,), (0,)), ((), ()))
    msk = m_ref[...]
    for b in range(bB):
        x64 = xt_ref[b].reshape(C * R, F)
        xe4 = jnp.dot(x64, e_ref[0], preferred_element_type=jnp.float32) * msk
        xo4 = jnp.dot(x64, e_ref[1], preferred_element_type=jnp.float32) * msk
        p = lax.dot_general(xe4, w_ref[0], dn,
                            preferred_element_type=jnp.float32)
        q = lax.dot_general(xo4, w_ref[1], dn,
                            preferred_element_type=jnp.float32)
        o_ref[b] = (p + q).reshape(F2, R, C * C)


def kernel(log_likelihoods):
    B, F, C, R = log_likelihoods.shape
    F2 = F // 2
    CR, CC = C * R, C * C
    xt = jnp.transpose(log_likelihoods, (0, 2, 3, 1))
    w, e, msk = _build_consts(C, R, F)
    bB = 8
    yt = pl.pallas_call(
        _product_body,
        grid=(B // bB,),
        in_specs=[
            pl.BlockSpec((bB, C, R, F), lambda b: (b, 0, 0, 0)),
            pl.BlockSpec((2, CR, CC), lambda b: (0, 0, 0)),
            pl.BlockSpec((2, F, R * F2), lambda b: (0, 0, 0)),
            pl.BlockSpec((CR, R * F2), lambda b: (0, 0)),
        ],
        out_specs=pl.BlockSpec((bB, F2, R, CC), lambda b: (b, 0, 0, 0)),
        out_shape=jax.ShapeDtypeStruct((B, F2, R, CC), jnp.float32),
        compiler_params=pltpu.CompilerParams(
            dimension_semantics=("arbitrary",),
        ),
    )(xt, w, e, msk)
    return jnp.transpose(yt, (0, 1, 3, 2))
